# Initial kernel scaffold; baseline (speedup 1.0000x reference)
import jax
import jax.numpy as jnp
from jax.experimental import pallas as pl


def kernel(x, adj, W1, b1, W2, b2, W3, b3, W4, b4, W5, b5, Wa, ba, Wh1, bh1, g1, be1, Wh2, bh2, g2, be2, Wh3, bh3):
    raise NotImplementedError("write your pallas kernel here")


# speedup vs baseline: 1.3006x; 1.3006x over previous
import functools

import jax
import jax.numpy as jnp
from jax.experimental import pallas as pl
from jax.experimental.pallas import tpu as pltpu

N = 2708
NP = 2816
F = 1433
FP = 1536
H = 128
NL = 5
R = 256
NB = NP // R
CAT = NL * H
INV = 1.0 / (1.0 + 1e-5) ** 0.5

def _proj_body(x_ref, wc_ref, o_ref):
    xv = x_ref[...]
    for l in range(NL):
        o_ref[l] = jnp.dot(xv, wc_ref[:, l * H:(l + 1) * H],
                           preferred_element_type=jnp.float32)


def _main_body(adj_ref, p_ref, wrec_ref, bgc_ref, wa_ref, ba_ref, wh1_ref,
               hv_ref, wh2_ref, wh3_ref, cat_ref, out_ref, s_ref, xb_ref):
    l = pl.program_id(0)
    r = pl.program_id(1)
    row = r * R

    @pl.when((l == 0) & (r == 0))
    def _init():
        s_ref[0] = p_ref[0]
        xb_ref[...] = jnp.zeros((NL - 1, NP, H), jnp.float32)

    cur = jax.lax.rem(l, 2)
    a_blk = adj_ref[pl.ds(row, R), :]
    acc = jnp.dot(a_blk, s_ref[cur], preferred_element_type=jnp.float32)
    xi = acc + bgc_ref[l, 0:1, :]
    xi = jnp.where(l == 0, jnp.maximum(xi, 0.0), xi)
    cat_ref[...] = xi

    @pl.when(l < NL - 1)
    def _advance():
        xb_ref[l, pl.ds(row, R), :] = xi
        xcat = jnp.concatenate(
            [xb_ref[j, pl.ds(row, R), :] for j in range(NL - 1)], axis=1)
        s_ref[1 - cur, pl.ds(row, R), :] = (
            p_ref[l + 1, pl.ds(row, R), :]
            + jnp.dot(xcat, wrec_ref[l], preferred_element_type=jnp.float32))

    @pl.when(l == NL - 1)
    def _head():
        catr = jnp.concatenate(
            [xb_ref[j, pl.ds(row, R), :] for j in range(NL - 1)] + [xi],
            axis=1)
        logits = jnp.dot(catr, wa_ref[...],
                         preferred_element_type=jnp.float32) + ba_ref[0:1, :]
        m = jnp.max(logits, axis=1, keepdims=True)
        e = jnp.exp(logits - m)
        aw = e / jnp.sum(e, axis=1, keepdims=True)
        att = catr * aw
        h = jnp.dot(att, wh1_ref[...],
                    preferred_element_type=jnp.float32) + hv_ref[0:1, :]
        h = jnp.maximum(hv_ref[1:2, :] * (h * INV) + hv_ref[2:3, :], 0.0)
        h2 = jnp.dot(h, wh2_ref[...],
                     preferred_element_type=jnp.float32) + hv_ref[3:4, :]
        h2 = jnp.maximum(hv_ref[4:5, :] * (h2 * INV) + hv_ref[5:6, :], 0.0)
        lg = jnp.dot(h2, wh3_ref[...],
                     preferred_element_type=jnp.float32) + hv_ref[6:7, :]
        col = jax.lax.broadcasted_iota(jnp.int32, (R, H), 1)
        mask = col < 7
        lgm = jnp.where(mask, lg, -1e30)
        mm = jnp.max(lgm, axis=1, keepdims=True)
        ee = jnp.where(mask, jnp.exp(lg - mm), 0.0)
        out_ref[...] = lg - mm - jnp.log(jnp.sum(ee, axis=1, keepdims=True))


def _pad_rows(a, rows):
    return jnp.pad(a, ((0, rows - a.shape[0]),) + ((0, 0),) * (a.ndim - 1))


def _pad_vec(v, n):
    return jnp.pad(v, (0, n - v.shape[0]))


@functools.partial(jax.jit)
def kernel(x, adj, W1, b1, W2, b2, W3, b3, W4, b4, W5, b5, Wa, ba,
           Wh1, bh1, g1, be1, Wh2, bh2, g2, be2, Wh3, bh3):
    xp = jnp.pad(x, ((0, NP - N), (0, FP - F)))
    adjp = jnp.pad(adj, ((0, NP - N), (0, NP - N)))
    Ws = [W1, W2, W3, W4, W5]
    wc = jnp.pad(jnp.concatenate([w[:F] for w in Ws], axis=1),
                 ((0, FP - F), (0, 0)))
    wrec = jnp.stack([_pad_rows(w[F:], (NL - 1) * H) for w in Ws[1:]])
    bgc = jnp.pad(jnp.stack([b1, b2, b3, b4, b5]).reshape(NL, 1, H),
                  ((0, 0), (0, 7), (0, 0)))
    ba_p = jnp.pad(ba.reshape(1, CAT), ((0, 7), (0, 0)))
    hv = jnp.pad(jnp.stack([bh1, g1, be1, _pad_vec(bh2, H), _pad_vec(g2, H),
                            _pad_vec(be2, H), _pad_vec(bh3, H)]),
                 ((0, 1), (0, 0)))
    wh2p = jnp.pad(Wh2, ((0, 0), (0, H - Wh2.shape[1])))
    wh3p = jnp.pad(Wh3, ((0, H - Wh3.shape[0]), (0, H - Wh3.shape[1])))

    p3 = pl.pallas_call(
        _proj_body,
        grid=(NB,),
        in_specs=[
            pl.BlockSpec((R, FP), lambda r: (r, 0)),
            pl.BlockSpec((FP, CAT), lambda r: (0, 0)),
        ],
        out_specs=pl.BlockSpec((NL, R, H), lambda r: (0, r, 0)),
        out_shape=jax.ShapeDtypeStruct((NL, NP, H), jnp.float32),
        compiler_params=pltpu.CompilerParams(
            dimension_semantics=("arbitrary",)),
    )(xp, wc)

    cat, outp = pl.pallas_call(
        _main_body,
        grid=(NL, NB),
        in_specs=[
            pl.BlockSpec((NP, NP), lambda l, r: (0, 0)),
            pl.BlockSpec((NL, NP, H), lambda l, r: (0, 0, 0)),
            pl.BlockSpec((NL - 1, (NL - 1) * H, H), lambda l, r: (0, 0, 0)),
            pl.BlockSpec((NL, 8, H), lambda l, r: (0, 0, 0)),
            pl.BlockSpec((CAT, CAT), lambda l, r: (0, 0)),
            pl.BlockSpec((8, CAT), lambda l, r: (0, 0)),
            pl.BlockSpec((CAT, H), lambda l, r: (0, 0)),
            pl.BlockSpec((8, H), lambda l, r: (0, 0)),
            pl.BlockSpec((H, H), lambda l, r: (0, 0)),
            pl.BlockSpec((H, H), lambda l, r: (0, 0)),
        ],
        out_specs=[
            pl.BlockSpec((R, H), lambda l, r: (r, l)),
            pl.BlockSpec((R, H), lambda l, r: (r, 0)),
        ],
        out_shape=[
            jax.ShapeDtypeStruct((NP, CAT), jnp.float32),
            jax.ShapeDtypeStruct((NP, H), jnp.float32),
        ],
        scratch_shapes=[
            pltpu.VMEM((2, NP, H), jnp.float32),
            pltpu.VMEM((NL - 1, NP, H), jnp.float32),
        ],
        compiler_params=pltpu.CompilerParams(
            dimension_semantics=("arbitrary", "arbitrary"),
            vmem_limit_bytes=128 * 1024 * 1024),
    )(adjp, p3, wrec, bgc, Wa, ba_p, Wh1, hv, wh2p, wh3p)

    features = cat[:N].reshape(N, NL, H)
    return outp[:N, :7], features


# speedup vs baseline: 1.3032x; 1.0020x over previous
import functools

import jax
import jax.numpy as jnp
from jax.experimental import pallas as pl
from jax.experimental.pallas import tpu as pltpu

N = 2708
NP = 2816
F = 1433
FP = 1536
H = 128
NL = 5
R = 256
NB = NP // R
CAT = NL * H
INV = 1.0 / (1.0 + 1e-5) ** 0.5

def _proj_body(x_ref, wc_ref, o_ref):
    xv = x_ref[...]
    for l in range(NL):
        o_ref[l] = jnp.dot(xv, wc_ref[:, l * H:(l + 1) * H],
                           preferred_element_type=jnp.float32)


def _main_body(adj_ref, p_ref, wrec_ref, bgc_ref, wa_ref, ba_ref, wh1_ref,
               hv_ref, wh2_ref, wh3_ref, cat_ref, out_ref, s_ref, xb_ref,
               adj_s_ref):
    l = pl.program_id(0)
    r = pl.program_id(1)
    row = r * R

    @pl.when((l == 0) & (r == 0))
    def _init():
        s_ref[0] = p_ref[0]
        xb_ref[...] = jnp.zeros((NL - 1, NP, H), jnp.float32)
        adj_s_ref[:, N:] = jnp.zeros((NP, NP - N), jnp.float32)

    @pl.when(l == 0)
    def _load_adj():
        aval = adj_ref[...]
        rmask = jax.lax.broadcasted_iota(jnp.int32, (R, N), 0) < (N - row)
        adj_s_ref[pl.ds(row, R), :N] = jnp.where(rmask, aval, 0.0)

    cur = jax.lax.rem(l, 2)
    a_blk = adj_s_ref[pl.ds(row, R), :]
    acc = jnp.dot(a_blk, s_ref[cur], preferred_element_type=jnp.float32)
    xi = acc + bgc_ref[l, 0:1, :]
    xi = jnp.where(l == 0, jnp.maximum(xi, 0.0), xi)
    cat_ref[...] = xi

    @pl.when(l < NL - 1)
    def _advance():
        xb_ref[l, pl.ds(row, R), :] = xi
        xcat = jnp.concatenate(
            [xb_ref[j, pl.ds(row, R), :] for j in range(NL - 1)], axis=1)
        s_ref[1 - cur, pl.ds(row, R), :] = (
            p_ref[l + 1, pl.ds(row, R), :]
            + jnp.dot(xcat, wrec_ref[l], preferred_element_type=jnp.float32))

    @pl.when(l == NL - 1)
    def _head():
        catr = jnp.concatenate(
            [xb_ref[j, pl.ds(row, R), :] for j in range(NL - 1)] + [xi],
            axis=1)
        logits = jnp.dot(catr, wa_ref[...],
                         preferred_element_type=jnp.float32) + ba_ref[0:1, :]
        m = jnp.max(logits, axis=1, keepdims=True)
        e = jnp.exp(logits - m)
        aw = e / jnp.sum(e, axis=1, keepdims=True)
        att = catr * aw
        h = jnp.dot(att, wh1_ref[...],
                    preferred_element_type=jnp.float32) + hv_ref[0:1, :]
        h = jnp.maximum(hv_ref[1:2, :] * (h * INV) + hv_ref[2:3, :], 0.0)
        h2 = jnp.dot(h, wh2_ref[...],
                     preferred_element_type=jnp.float32) + hv_ref[3:4, :]
        h2 = jnp.maximum(hv_ref[4:5, :] * (h2 * INV) + hv_ref[5:6, :], 0.0)
        lg = jnp.dot(h2, wh3_ref[...],
                     preferred_element_type=jnp.float32) + hv_ref[6:7, :]
        col = jax.lax.broadcasted_iota(jnp.int32, (R, H), 1)
        mask = col < 7
        lgm = jnp.where(mask, lg, -1e30)
        mm = jnp.max(lgm, axis=1, keepdims=True)
        ee = jnp.where(mask, jnp.exp(lg - mm), 0.0)
        out_ref[...] = lg - mm - jnp.log(jnp.sum(ee, axis=1, keepdims=True))


def _pad_rows(a, rows):
    return jnp.pad(a, ((0, rows - a.shape[0]),) + ((0, 0),) * (a.ndim - 1))


def _pad_vec(v, n):
    return jnp.pad(v, (0, n - v.shape[0]))


@functools.partial(jax.jit)
def kernel(x, adj, W1, b1, W2, b2, W3, b3, W4, b4, W5, b5, Wa, ba,
           Wh1, bh1, g1, be1, Wh2, bh2, g2, be2, Wh3, bh3):
    xp = jnp.pad(x, ((0, NP - N), (0, FP - F)))
    Ws = [W1, W2, W3, W4, W5]
    wc = jnp.pad(jnp.concatenate([w[:F] for w in Ws], axis=1),
                 ((0, FP - F), (0, 0)))
    wrec = jnp.stack([_pad_rows(w[F:], (NL - 1) * H) for w in Ws[1:]])
    bgc = jnp.pad(jnp.stack([b1, b2, b3, b4, b5]).reshape(NL, 1, H),
                  ((0, 0), (0, 7), (0, 0)))
    ba_p = jnp.pad(ba.reshape(1, CAT), ((0, 7), (0, 0)))
    hv = jnp.pad(jnp.stack([bh1, g1, be1, _pad_vec(bh2, H), _pad_vec(g2, H),
                            _pad_vec(be2, H), _pad_vec(bh3, H)]),
                 ((0, 1), (0, 0)))
    wh2p = jnp.pad(Wh2, ((0, 0), (0, H - Wh2.shape[1])))
    wh3p = jnp.pad(Wh3, ((0, H - Wh3.shape[0]), (0, H - Wh3.shape[1])))

    p3 = pl.pallas_call(
        _proj_body,
        grid=(NB,),
        in_specs=[
            pl.BlockSpec((R, FP), lambda r: (r, 0)),
            pl.BlockSpec((FP, CAT), lambda r: (0, 0)),
        ],
        out_specs=pl.BlockSpec((NL, R, H), lambda r: (0, r, 0)),
        out_shape=jax.ShapeDtypeStruct((NL, NP, H), jnp.float32),
        compiler_params=pltpu.CompilerParams(
            dimension_semantics=("arbitrary",)),
    )(xp, wc)

    cat, outp = pl.pallas_call(
        _main_body,
        grid=(NL, NB),
        in_specs=[
            pl.BlockSpec((R, N), lambda l, r: (jnp.where(l == 0, r, 0), 0)),
            pl.BlockSpec((NL, NP, H), lambda l, r: (0, 0, 0)),
            pl.BlockSpec((NL - 1, (NL - 1) * H, H), lambda l, r: (0, 0, 0)),
            pl.BlockSpec((NL, 8, H), lambda l, r: (0, 0, 0)),
            pl.BlockSpec((CAT, CAT), lambda l, r: (0, 0)),
            pl.BlockSpec((8, CAT), lambda l, r: (0, 0)),
            pl.BlockSpec((CAT, H), lambda l, r: (0, 0)),
            pl.BlockSpec((8, H), lambda l, r: (0, 0)),
            pl.BlockSpec((H, H), lambda l, r: (0, 0)),
            pl.BlockSpec((H, H), lambda l, r: (0, 0)),
        ],
        out_specs=[
            pl.BlockSpec((R, H), lambda l, r: (r, l)),
            pl.BlockSpec((R, H), lambda l, r: (r, 0)),
        ],
        out_shape=[
            jax.ShapeDtypeStruct((NP, CAT), jnp.float32),
            jax.ShapeDtypeStruct((NP, H), jnp.float32),
        ],
        scratch_shapes=[
            pltpu.VMEM((2, NP, H), jnp.float32),
            pltpu.VMEM((NL - 1, NP, H), jnp.float32),
            pltpu.VMEM((NP, NP), jnp.float32),
        ],
        compiler_params=pltpu.CompilerParams(
            dimension_semantics=("arbitrary", "arbitrary"),
            vmem_limit_bytes=128 * 1024 * 1024),
    )(adj, p3, wrec, bgc, Wa, ba_p, Wh1, hv, wh2p, wh3p)

    features = cat[:N].reshape(N, NL, H)
    return outp[:N, :7], features


# speedup vs baseline: 2.1164x; 1.6240x over previous
import functools

import jax
import jax.numpy as jnp
from jax.experimental import pallas as pl
from jax.experimental.pallas import tpu as pltpu

N = 2708
NP = 2816
F = 1433
FP = 1536
H = 128
NL = 5
R = 256
NB = NP // R
CAT = NL * H
INV = 1.0 / (1.0 + 1e-5) ** 0.5

def _proj_body(x_ref, wc_ref, o_ref):
    row = pl.program_id(0) * R
    rmask = jax.lax.broadcasted_iota(jnp.int32, (R, F), 0) < (N - row)
    xv = jnp.where(rmask, x_ref[...], 0.0)
    for l in range(NL):
        o_ref[l] = jnp.dot(xv, wc_ref[:, l * H:(l + 1) * H],
                           preferred_element_type=jnp.float32)


def _main_body(adj_ref, p_ref, wrec_ref, bgc_ref, wa_ref, ba_ref, wh1_ref,
               hv_ref, wh2_ref, wh3_ref, cat_ref, out_ref, s_ref, xb_ref,
               adj_s_ref):
    l = pl.program_id(0)
    r = pl.program_id(1)
    row = r * R

    @pl.when((l == 0) & (r == 0))
    def _init():
        s_ref[0] = p_ref[0]
        xb_ref[...] = jnp.zeros((NL - 1, NP, H), jnp.float32)
        adj_s_ref[:, N:] = jnp.zeros((NP, NP - N), jnp.float32)

    @pl.when(l == 0)
    def _load_adj():
        aval = adj_ref[...]
        rmask = jax.lax.broadcasted_iota(jnp.int32, (R, N), 0) < (N - row)
        adj_s_ref[pl.ds(row, R), :N] = jnp.where(rmask, aval, 0.0)

    cur = jax.lax.rem(l, 2)
    a_blk = adj_s_ref[pl.ds(row, R), :]
    acc = jnp.dot(a_blk, s_ref[cur], preferred_element_type=jnp.float32)
    xi = acc + bgc_ref[l, 0:1, :]
    xi = jnp.where(l == 0, jnp.maximum(xi, 0.0), xi)
    cat_ref[...] = xi

    @pl.when(l < NL - 1)
    def _advance():
        xb_ref[l, pl.ds(row, R), :] = xi
        xcat = jnp.concatenate(
            [xb_ref[j, pl.ds(row, R), :] for j in range(NL - 1)], axis=1)
        s_ref[1 - cur, pl.ds(row, R), :] = (
            p_ref[l + 1, pl.ds(row, R), :]
            + jnp.dot(xcat, wrec_ref[l], preferred_element_type=jnp.float32))

    @pl.when(l == NL - 1)
    def _head():
        catr = jnp.concatenate(
            [xb_ref[j, pl.ds(row, R), :] for j in range(NL - 1)] + [xi],
            axis=1)
        logits = jnp.dot(catr, wa_ref[...],
                         preferred_element_type=jnp.float32) + ba_ref[0:1, :]
        m = jnp.max(logits, axis=1, keepdims=True)
        e = jnp.exp(logits - m)
        aw = e / jnp.sum(e, axis=1, keepdims=True)
        att = catr * aw
        h = jnp.dot(att, wh1_ref[...],
                    preferred_element_type=jnp.float32) + hv_ref[0:1, :]
        h = jnp.maximum(hv_ref[1:2, :] * (h * INV) + hv_ref[2:3, :], 0.0)
        h2 = jnp.dot(h, wh2_ref[...],
                     preferred_element_type=jnp.float32) + hv_ref[3:4, :]
        h2 = jnp.maximum(hv_ref[4:5, :] * (h2 * INV) + hv_ref[5:6, :], 0.0)
        lg = jnp.dot(h2, wh3_ref[...],
                     preferred_element_type=jnp.float32) + hv_ref[6:7, :]
        col = jax.lax.broadcasted_iota(jnp.int32, (R, H), 1)
        mask = col < 7
        lgm = jnp.where(mask, lg, -1e30)
        mm = jnp.max(lgm, axis=1, keepdims=True)
        ee = jnp.where(mask, jnp.exp(lg - mm), 0.0)
        out_ref[...] = lg - mm - jnp.log(jnp.sum(ee, axis=1, keepdims=True))


def _pad_rows(a, rows):
    return jnp.pad(a, ((0, rows - a.shape[0]),) + ((0, 0),) * (a.ndim - 1))


def _pad_vec(v, n):
    return jnp.pad(v, (0, n - v.shape[0]))


@functools.partial(jax.jit)
def kernel(x, adj, W1, b1, W2, b2, W3, b3, W4, b4, W5, b5, Wa, ba,
           Wh1, bh1, g1, be1, Wh2, bh2, g2, be2, Wh3, bh3):
    Ws = [W1, W2, W3, W4, W5]
    wc = jnp.concatenate([w[:F] for w in Ws], axis=1)
    wrec = jnp.stack([_pad_rows(w[F:], (NL - 1) * H) for w in Ws[1:]])
    bgc = jnp.pad(jnp.stack([b1, b2, b3, b4, b5]).reshape(NL, 1, H),
                  ((0, 0), (0, 7), (0, 0)))
    ba_p = jnp.pad(ba.reshape(1, CAT), ((0, 7), (0, 0)))
    hv = jnp.pad(jnp.stack([bh1, g1, be1, _pad_vec(bh2, H), _pad_vec(g2, H),
                            _pad_vec(be2, H), _pad_vec(bh3, H)]),
                 ((0, 1), (0, 0)))
    wh2p = jnp.pad(Wh2, ((0, 0), (0, H - Wh2.shape[1])))
    wh3p = jnp.pad(Wh3, ((0, H - Wh3.shape[0]), (0, H - Wh3.shape[1])))

    p3 = pl.pallas_call(
        _proj_body,
        grid=(NB,),
        in_specs=[
            pl.BlockSpec((R, F), lambda r: (r, 0)),
            pl.BlockSpec((F, CAT), lambda r: (0, 0)),
        ],
        out_specs=pl.BlockSpec((NL, R, H), lambda r: (0, r, 0)),
        out_shape=jax.ShapeDtypeStruct((NL, NP, H), jnp.float32),
        compiler_params=pltpu.CompilerParams(
            dimension_semantics=("arbitrary",)),
    )(x, wc)

    cat, outp = pl.pallas_call(
        _main_body,
        grid=(NL, NB),
        in_specs=[
            pl.BlockSpec((R, N), lambda l, r: (jnp.where(l == 0, r, 0), 0)),
            pl.BlockSpec((NL, NP, H), lambda l, r: (0, 0, 0)),
            pl.BlockSpec((NL - 1, (NL - 1) * H, H), lambda l, r: (0, 0, 0)),
            pl.BlockSpec((NL, 8, H), lambda l, r: (0, 0, 0)),
            pl.BlockSpec((CAT, CAT), lambda l, r: (0, 0)),
            pl.BlockSpec((8, CAT), lambda l, r: (0, 0)),
            pl.BlockSpec((CAT, H), lambda l, r: (0, 0)),
            pl.BlockSpec((8, H), lambda l, r: (0, 0)),
            pl.BlockSpec((H, H), lambda l, r: (0, 0)),
            pl.BlockSpec((H, H), lambda l, r: (0, 0)),
        ],
        out_specs=[
            pl.BlockSpec((R, H), lambda l, r: (r, l)),
            pl.BlockSpec((R, H), lambda l, r: (r, 0)),
        ],
        out_shape=[
            jax.ShapeDtypeStruct((N, CAT), jnp.float32),
            jax.ShapeDtypeStruct((N, H), jnp.float32),
        ],
        scratch_shapes=[
            pltpu.VMEM((2, NP, H), jnp.float32),
            pltpu.VMEM((NL - 1, NP, H), jnp.float32),
            pltpu.VMEM((NP, NP), jnp.float32),
        ],
        compiler_params=pltpu.CompilerParams(
            dimension_semantics=("arbitrary", "arbitrary"),
            vmem_limit_bytes=128 * 1024 * 1024),
    )(adj, p3, wrec, bgc, Wa, ba_p, Wh1, hv, wh2p, wh3p)

    features = cat.reshape(N, NL, H)
    return outp[:, :7], features


# speedup vs baseline: 2.2154x; 1.0468x over previous
import functools

import jax
import jax.numpy as jnp
from jax.experimental import pallas as pl
from jax.experimental.pallas import tpu as pltpu

N = 2708
NP = 2816
F = 1433
H = 128
NL = 5
R = 256
NB = NP // R
CAT = NL * H
INV = 1.0 / (1.0 + 1e-5) ** 0.5


def _body(adj_ref, x_ref, w1_ref, w2_ref, w3_ref, w4_ref, w5_ref,
          bgc_ref, wa_ref, ba_ref,
          wh1_ref, hv_ref, wh2_ref, wh3_ref, cat_ref, out_ref,
          p_ref, s_ref, xb_ref, adj_s_ref):
    p = pl.program_id(0)
    r = pl.program_id(1)
    row = r * R

    @pl.when(p == 0)
    def _load():
        @pl.when(r == 0)
        def _zero_pad_cols():
            adj_s_ref[:, N:] = jnp.zeros((NP, NP - N), jnp.float32)

        rmask = jax.lax.broadcasted_iota(jnp.int32, (R, N), 0) < (N - row)
        adj_s_ref[pl.ds(row, R), :N] = jnp.where(rmask, adj_ref[...], 0.0)

        xmask = jax.lax.broadcasted_iota(jnp.int32, (R, F), 0) < (N - row)
        xv = jnp.where(xmask, x_ref[...], 0.0)
        ws = [w1_ref, w2_ref, w3_ref, w4_ref, w5_ref]
        s_ref[0, pl.ds(row, R), :] = jnp.dot(
            xv, ws[0][...], preferred_element_type=jnp.float32)
        for l in range(1, NL):
            p_ref[l - 1, pl.ds(row, R), :] = jnp.dot(
                xv, ws[l][:F, :], preferred_element_type=jnp.float32)

    @pl.when(p > 0)
    def _layer():
        l = p - 1
        cur = jax.lax.rem(l, 2)
        a_blk = adj_s_ref[pl.ds(row, R), :]
        acc = jnp.dot(a_blk, s_ref[cur], preferred_element_type=jnp.float32)
        xi = acc + bgc_ref[l, 0:1, :]
        xi = jnp.where(l == 0, jnp.maximum(xi, 0.0), xi)
        cat_ref[...] = xi

        wrs = [w2_ref, w3_ref, w4_ref, w5_ref]
        for ll in range(NL - 1):
            @pl.when(l == ll)
            def _advance(ll=ll):
                xb_ref[ll, pl.ds(row, R), :] = xi
                xcat = jnp.concatenate(
                    [xb_ref[j, pl.ds(row, R), :] for j in range(ll + 1)],
                    axis=1) if ll > 0 else xi
                s_ref[1 - cur, pl.ds(row, R), :] = (
                    p_ref[ll, pl.ds(row, R), :]
                    + jnp.dot(xcat, wrs[ll][F:, :],
                              preferred_element_type=jnp.float32))

        @pl.when(l == NL - 1)
        def _head():
            catr = jnp.concatenate(
                [xb_ref[j, pl.ds(row, R), :] for j in range(NL - 1)] + [xi],
                axis=1)
            logits = jnp.dot(catr, wa_ref[...],
                             preferred_element_type=jnp.float32) + ba_ref[0:1, :]
            m = jnp.max(logits, axis=1, keepdims=True)
            e = jnp.exp(logits - m)
            aw = e / jnp.sum(e, axis=1, keepdims=True)
            att = catr * aw
            h = jnp.dot(att, wh1_ref[...],
                        preferred_element_type=jnp.float32) + hv_ref[0:1, :]
            h = jnp.maximum(hv_ref[1:2, :] * (h * INV) + hv_ref[2:3, :], 0.0)
            h2 = jnp.dot(h, wh2_ref[...],
                         preferred_element_type=jnp.float32) + hv_ref[3:4, :]
            h2 = jnp.maximum(hv_ref[4:5, :] * (h2 * INV) + hv_ref[5:6, :], 0.0)
            lg = jnp.dot(h2, wh3_ref[...],
                         preferred_element_type=jnp.float32) + hv_ref[6:7, :]
            col = jax.lax.broadcasted_iota(jnp.int32, (R, H), 1)
            mask = col < 7
            lgm = jnp.where(mask, lg, -1e30)
            mm = jnp.max(lgm, axis=1, keepdims=True)
            ee = jnp.where(mask, jnp.exp(lg - mm), 0.0)
            out_ref[...] = lg - mm - jnp.log(jnp.sum(ee, axis=1, keepdims=True))


def _pad_vec(v, n):
    return jnp.pad(v, (0, n - v.shape[0]))


@functools.partial(jax.jit)
def kernel(x, adj, W1, b1, W2, b2, W3, b3, W4, b4, W5, b5, Wa, ba,
           Wh1, bh1, g1, be1, Wh2, bh2, g2, be2, Wh3, bh3):
    bgc = jnp.pad(jnp.stack([b1, b2, b3, b4, b5]).reshape(NL, 1, H),
                  ((0, 0), (0, 7), (0, 0)))
    ba_p = jnp.pad(ba.reshape(1, CAT), ((0, 7), (0, 0)))
    hv = jnp.pad(jnp.stack([bh1, g1, be1, _pad_vec(bh2, H), _pad_vec(g2, H),
                            _pad_vec(be2, H), _pad_vec(bh3, H)]),
                 ((0, 1), (0, 0)))
    wh2p = jnp.pad(Wh2, ((0, 0), (0, H - Wh2.shape[1])))
    wh3p = jnp.pad(Wh3, ((0, H - Wh3.shape[0]), (0, H - Wh3.shape[1])))

    full = lambda shape: pl.BlockSpec(shape, lambda p, r: (0,) * len(shape))
    stream = pl.BlockSpec((R, N), lambda p, r: (jnp.where(p == 0, r, 0), 0))
    streamx = pl.BlockSpec((R, F), lambda p, r: (jnp.where(p == 0, r, 0), 0))

    cat, outp = pl.pallas_call(
        _body,
        grid=(NL + 1, NB),
        in_specs=[
            stream,
            streamx,
            full((F, H)), full((F + H, H)), full((F + 2 * H, H)),
            full((F + 3 * H, H)), full((F + 4 * H, H)),
            full((NL, 8, H)),
            full((CAT, CAT)),
            full((8, CAT)),
            full((CAT, H)),
            full((8, H)),
            full((H, H)),
            full((H, H)),
        ],
        out_specs=[
            pl.BlockSpec((R, H), lambda p, r: (jnp.where(p == 0, 0, r),
                                               jnp.maximum(p - 1, 0))),
            pl.BlockSpec((R, H), lambda p, r: (jnp.where(p == NL, r, 0), 0)),
        ],
        out_shape=[
            jax.ShapeDtypeStruct((N, CAT), jnp.float32),
            jax.ShapeDtypeStruct((N, H), jnp.float32),
        ],
        scratch_shapes=[
            pltpu.VMEM((NL - 1, NP, H), jnp.float32),
            pltpu.VMEM((2, NP, H), jnp.float32),
            pltpu.VMEM((NL - 1, NP, H), jnp.float32),
            pltpu.VMEM((NP, NP), jnp.float32),
        ],
        compiler_params=pltpu.CompilerParams(
            dimension_semantics=("arbitrary", "arbitrary"),
            vmem_limit_bytes=128 * 1024 * 1024),
    )(adj, x, W1, W2, W3, W4, W5, bgc, Wa, ba_p, Wh1, hv, wh2p, wh3p)

    features = cat.reshape(N, NL, H)
    return outp[:, :7], features
